# baseline (speedup 1.0000x reference)
import functools

import jax
import jax.numpy as jnp
from jax import lax
from jax.experimental import pallas as pl
from jax.experimental.pallas import tpu as pltpu
from jax.experimental.pallas import tpu_sc as plsc

BATCH = 16384
D = 64
OFFSET = 100000
NC = 2
NS = 16
L = 16
NW = NC * NS
BPW = BATCH // NW
CHUNK = 128
NCHUNK = BPW // CHUNK

_mesh = plsc.VectorSubcoreMesh(core_axis_name="c", subcore_axis_name="s")


@functools.partial(
    pl.kernel,
    mesh=_mesh,
    compiler_params=pltpu.CompilerParams(
        needs_layout_passes=False, use_tc_tiling_on_sc=False),
    out_type=jax.ShapeDtypeStruct((BATCH,), jnp.float32),
    scratch_types=[
        pltpu.VMEM((2 * BPW,), jnp.int32),
        pltpu.VMEM((NCHUNK, CHUNK), jnp.int32),
        pltpu.VMEM((NCHUNK, CHUNK), jnp.int32),
        pltpu.VMEM((BPW, D), jnp.float32),
        pltpu.VMEM((BPW, D), jnp.float32),
        pltpu.VMEM((80,), jnp.float32),
        pltpu.VMEM((BPW,), jnp.float32),
        pltpu.SemaphoreType.DMA,
    ],
)
def _gmf_sc(x_hbm, table_hbm, params_hbm, out_hbm,
            xv, uidx, vidx, urows, vrows, pv, outv, sem):
    wid = lax.axis_index("s") * NC + lax.axis_index("c")
    base = wid * BPW

    pltpu.sync_copy(x_hbm.at[pl.ds(2 * base, 2 * BPW)], xv)
    pltpu.sync_copy(params_hbm, pv)

    lanes2 = lax.iota(jnp.int32, L) * 2
    for g in range(BPW // L):
        u = plsc.load_gather(xv, [lanes2 + (2 * L * g)])
        v = plsc.load_gather(xv, [lanes2 + (2 * L * g + 1)]) + OFFSET
        r, c = g // (CHUNK // L), (g % (CHUNK // L)) * L
        uidx[r, pl.ds(c, L)] = u
        vidx[r, pl.ds(c, L)] = v

    copies = []
    for j in range(NCHUNK):
        copies.append(pltpu.async_copy(
            table_hbm.at[uidx.at[j]], urows.at[pl.ds(j * CHUNK, CHUNK)], sem))
        copies.append(pltpu.async_copy(
            table_hbm.at[vidx.at[j]], vrows.at[pl.ds(j * CHUNK, CHUNK)], sem))
    for cp in copies:
        cp.wait()

    w0 = pv[pl.ds(0, L)]
    w1 = pv[pl.ds(16, L)]
    w2 = pv[pl.ds(32, L)]
    w3 = pv[pl.ds(48, L)]
    bias = pv[pl.ds(64, L)][0]
    lanes = lax.iota(jnp.int32, L)

    def group(g, carry):
        res = jnp.zeros((L,), jnp.float32)
        for j in range(L):
            i = g * L + j
            acc = (urows[i, pl.ds(0, L)] * vrows[i, pl.ds(0, L)]) * w0
            acc = acc + (urows[i, pl.ds(16, L)] * vrows[i, pl.ds(16, L)]) * w1
            acc = acc + (urows[i, pl.ds(32, L)] * vrows[i, pl.ds(32, L)]) * w2
            acc = acc + (urows[i, pl.ds(48, L)] * vrows[i, pl.ds(48, L)]) * w3
            res = jnp.where(lanes == j, jnp.sum(acc), res)
        outv[pl.ds(g * L, L)] = jnp.maximum(res + bias, 0.0)
        return carry

    lax.fori_loop(0, BPW // L, group, 0)

    pltpu.sync_copy(outv, out_hbm.at[pl.ds(base, BPW)])


def kernel(x, table, W, b):
    xflat = x.astype(jnp.int32).reshape(2 * BATCH)
    params = jnp.concatenate(
        [W.reshape(D).astype(jnp.float32), b.astype(jnp.float32),
         jnp.zeros((15,), jnp.float32)])
    out = _gmf_sc(xflat, table, params)
    return out.reshape(BATCH, 1)
